# baseline (speedup 1.0000x reference)
import functools

import jax
import jax.numpy as jnp
from jax import lax
from jax.experimental import pallas as pl
from jax.experimental.pallas import tpu as pltpu
from jax.experimental.pallas import tpu_sc as plsc

VOCAB_ROWS = 1000001
EMBED_DIM = 64
BATCH = 16384

_CHUNK_C = 40960
_GRID_A = -(-VOCAB_ROWS // _CHUNK_C)
_QUARTER = _CHUNK_C // 4
_PACK_ROWS = _GRID_A * _QUARTER

_NC = 2
_NS = 16
_NW = _NC * _NS
_B_PER_W = BATCH // _NW
_IDX_CHUNK = 128
_N_CHUNKS = _B_PER_W // _IDX_CHUNK


def _bf16_lo(v):
    return (v + jnp.uint32(0x7FFF) + ((v >> 16) & jnp.uint32(1))) >> 16


def _bf16_hi(v):
    return (v + jnp.uint32(0x7FFF) + ((v >> 16) & jnp.uint32(1))) & jnp.uint32(
        0xFFFF0000
    )


def _pack_block(tT_ref, out_ref):
    x = tT_ref[...]
    xt = jnp.swapaxes(x, 0, 1)
    u = lax.bitcast_convert_type(xt, jnp.uint32)
    a = u[0 * _QUARTER : 1 * _QUARTER]
    b = u[1 * _QUARTER : 2 * _QUARTER]
    c = u[2 * _QUARTER : 3 * _QUARTER]
    d = u[3 * _QUARTER : 4 * _QUARTER]
    p01 = _bf16_lo(a) | _bf16_hi(b)
    p23 = _bf16_lo(c) | _bf16_hi(d)
    out_ref[:, :EMBED_DIM] = lax.bitcast_convert_type(p01, jnp.float32)
    out_ref[:, EMBED_DIM:] = lax.bitcast_convert_type(p23, jnp.float32)


def _tc_pack(tableT):
    return pl.pallas_call(
        _pack_block,
        grid=(_GRID_A,),
        in_specs=[pl.BlockSpec((EMBED_DIM, _CHUNK_C), lambda i: (0, i))],
        out_specs=pl.BlockSpec((_QUARTER, 128), lambda i: (i, 0)),
        out_shape=jax.ShapeDtypeStruct((_PACK_ROWS, 128), jnp.float32),
    )(tableT)


def _sc_gather(packed, slot3d):
    mesh = plsc.VectorSubcoreMesh(core_axis_name="c", subcore_axis_name="s")

    @functools.partial(
        pl.kernel,
        mesh=mesh,
        compiler_params=pltpu.CompilerParams(use_tc_tiling_on_sc=True),
        out_type=jax.ShapeDtypeStruct((BATCH, 128), jnp.float32),
        scratch_types=[
            pltpu.VMEM((_N_CHUNKS, _IDX_CHUNK), jnp.int32),
            pltpu.VMEM((_B_PER_W, 128), jnp.float32),
            pltpu.SemaphoreType.DMA,
        ],
    )
    def gather_kernel(packed_hbm, idx_hbm, out_hbm, idx_v, rows_v, sem):
        wid = lax.axis_index("s") * _NC + lax.axis_index("c")
        base = wid * _B_PER_W
        pltpu.sync_copy(idx_hbm.at[wid], idx_v)
        copies = []
        for j in range(_N_CHUNKS):
            copies.append(
                pltpu.async_copy(
                    packed_hbm.at[idx_v.at[j]],
                    rows_v.at[pl.ds(j * _IDX_CHUNK, _IDX_CHUNK)],
                    sem,
                )
            )
        for c in copies:
            c.wait()
        pltpu.sync_copy(rows_v, out_hbm.at[pl.ds(base, _B_PER_W)])

    return gather_kernel(packed, slot3d)


def _mlp_block(x_ref, uid_ref, w1_ref, b1_ref, w2_ref, b2_ref,
               w3_ref, b3_ref, o_ref):
    x = lax.bitcast_convert_type(x_ref[...], jnp.uint32)
    sub = (uid_ref[...] % _CHUNK_C) // _QUARTER
    half = jnp.where(sub >= 2, x[:, EMBED_DIM:], x[:, :EMBED_DIM])
    shift = ((sub & 1) * 16).astype(jnp.uint32)
    bits = (half >> shift) & jnp.uint32(0xFFFF)
    emb = lax.bitcast_convert_type(
        bits.astype(jnp.uint16), jnp.bfloat16
    ).astype(jnp.float32)
    embT = jnp.swapaxes(emb, 0, 1)
    dn = (((0,), (0,)), ((), ()))
    h = jnp.maximum(
        lax.dot_general(w1_ref[...], embT, dn,
                        preferred_element_type=jnp.float32)
        + b1_ref[...],
        0.0,
    )
    h = jnp.maximum(
        lax.dot_general(w2_ref[...], h, dn,
                        preferred_element_type=jnp.float32)
        + b2_ref[...],
        0.0,
    )
    o_ref[...] = (
        lax.dot_general(w3_ref[...], h, dn,
                        preferred_element_type=jnp.float32)
        + b3_ref[...]
    )


def _tc_mlp(x, uid2d, W1, b1, W2, b2, W3, b3):
    blk = 4096
    grid = (BATCH // blk,)
    return pl.pallas_call(
        _mlp_block,
        grid=grid,
        in_specs=[
            pl.BlockSpec((blk, 128), lambda i: (i, 0)),
            pl.BlockSpec((blk, 1), lambda i: (i, 0)),
            pl.BlockSpec(W1.shape, lambda i: (0, 0)),
            pl.BlockSpec(b1.shape, lambda i: (0, 0)),
            pl.BlockSpec(W2.shape, lambda i: (0, 0)),
            pl.BlockSpec(b2.shape, lambda i: (0, 0)),
            pl.BlockSpec(W3.shape, lambda i: (0, 0)),
            pl.BlockSpec(b3.shape, lambda i: (0, 0)),
        ],
        out_specs=pl.BlockSpec((W3.shape[1], blk), lambda i: (0, i)),
        out_shape=jax.ShapeDtypeStruct((W3.shape[1], BATCH), jnp.float32),
    )(x, uid2d, W1, b1, W2, b2, W3, b3)


def kernel(user_id, table, W1, b1, W2, b2, W3, b3):
    uid = user_id.astype(jnp.int32)
    chunk = uid // _CHUNK_C
    r = uid % _CHUNK_C
    slot3d = (chunk * _QUARTER + r % _QUARTER).reshape(
        _NW, _N_CHUNKS, _IDX_CHUNK
    )
    packed = _tc_pack(table.T)
    rows = _sc_gather(packed, slot3d)
    outT = _tc_mlp(
        rows,
        uid.reshape(BATCH, 1),
        W1,
        b1.reshape(-1, 1),
        W2,
        b2.reshape(-1, 1),
        W3,
        b3.reshape(-1, 1),
    )
    return outT.T


# speedup vs baseline: 1.0172x; 1.0172x over previous
import functools

import jax
import jax.numpy as jnp
from jax import lax
from jax.experimental import pallas as pl
from jax.experimental.pallas import tpu as pltpu
from jax.experimental.pallas import tpu_sc as plsc

VOCAB_ROWS = 1000001
EMBED_DIM = 64
BATCH = 16384

_CHUNK_C = 32768
_GRID_A = -(-VOCAB_ROWS // _CHUNK_C)
_QUARTER = _CHUNK_C // 4
_PACK_ROWS = _GRID_A * _QUARTER

_NC = 2
_NS = 16
_NW = _NC * _NS
_B_PER_W = BATCH // _NW
_IDX_CHUNK = 128
_N_CHUNKS = _B_PER_W // _IDX_CHUNK


def _bf16_lo(v):
    return (v + jnp.uint32(0x7FFF) + ((v >> 16) & jnp.uint32(1))) >> 16


def _bf16_hi(v):
    return (v + jnp.uint32(0x7FFF) + ((v >> 16) & jnp.uint32(1))) & jnp.uint32(
        0xFFFF0000
    )


def _pack_block(tT_ref, out_ref):
    x = tT_ref[...]
    xt = jnp.swapaxes(x, 0, 1)
    u = lax.bitcast_convert_type(xt, jnp.uint32)
    a = u[0 * _QUARTER : 1 * _QUARTER]
    b = u[1 * _QUARTER : 2 * _QUARTER]
    c = u[2 * _QUARTER : 3 * _QUARTER]
    d = u[3 * _QUARTER : 4 * _QUARTER]
    p01 = _bf16_lo(a) | _bf16_hi(b)
    p23 = _bf16_lo(c) | _bf16_hi(d)
    out_ref[:, :EMBED_DIM] = lax.bitcast_convert_type(p01, jnp.float32)
    out_ref[:, EMBED_DIM:] = lax.bitcast_convert_type(p23, jnp.float32)


def _tc_pack(tableT):
    return pl.pallas_call(
        _pack_block,
        grid=(_GRID_A,),
        in_specs=[pl.BlockSpec((EMBED_DIM, _CHUNK_C), lambda i: (0, i))],
        out_specs=pl.BlockSpec((_QUARTER, 128), lambda i: (i, 0)),
        out_shape=jax.ShapeDtypeStruct((_PACK_ROWS, 128), jnp.float32),
    )(tableT)


def _sc_gather(packed, slot3d):
    mesh = plsc.VectorSubcoreMesh(core_axis_name="c", subcore_axis_name="s")

    @functools.partial(
        pl.kernel,
        mesh=mesh,
        compiler_params=pltpu.CompilerParams(use_tc_tiling_on_sc=True),
        out_type=jax.ShapeDtypeStruct((BATCH, 128), jnp.float32),
        scratch_types=[
            pltpu.VMEM((_N_CHUNKS, _IDX_CHUNK), jnp.int32),
            pltpu.VMEM((_B_PER_W, 128), jnp.float32),
            pltpu.SemaphoreType.DMA,
        ],
    )
    def gather_kernel(packed_hbm, idx_hbm, out_hbm, idx_v, rows_v, sem):
        wid = lax.axis_index("s") * _NC + lax.axis_index("c")
        base = wid * _B_PER_W
        pltpu.sync_copy(idx_hbm.at[wid], idx_v)
        copies = []
        for j in range(_N_CHUNKS):
            copies.append(
                pltpu.async_copy(
                    packed_hbm.at[idx_v.at[j]],
                    rows_v.at[pl.ds(j * _IDX_CHUNK, _IDX_CHUNK)],
                    sem,
                )
            )
        for c in copies:
            c.wait()
        pltpu.sync_copy(rows_v, out_hbm.at[pl.ds(base, _B_PER_W)])

    return gather_kernel(packed, slot3d)


def _mlp_block(x_ref, uid_ref, w1_ref, b1_ref, w2_ref, b2_ref,
               w3_ref, b3_ref, o_ref):
    x = lax.bitcast_convert_type(x_ref[...], jnp.uint32)
    sub = (uid_ref[...] % _CHUNK_C) // _QUARTER
    half = jnp.where(sub >= 2, x[:, EMBED_DIM:], x[:, :EMBED_DIM])
    shift = ((sub & 1) * 16).astype(jnp.uint32)
    bits = (half >> shift) & jnp.uint32(0xFFFF)
    emb = lax.bitcast_convert_type(
        bits.astype(jnp.uint16), jnp.bfloat16
    ).astype(jnp.float32)
    embT = jnp.swapaxes(emb, 0, 1)
    dn = (((0,), (0,)), ((), ()))
    h = jnp.maximum(
        lax.dot_general(w1_ref[...], embT, dn,
                        preferred_element_type=jnp.float32)
        + b1_ref[...],
        0.0,
    )
    h = jnp.maximum(
        lax.dot_general(w2_ref[...], h, dn,
                        preferred_element_type=jnp.float32)
        + b2_ref[...],
        0.0,
    )
    o_ref[...] = (
        lax.dot_general(w3_ref[...], h, dn,
                        preferred_element_type=jnp.float32)
        + b3_ref[...]
    )


def _tc_mlp(x, uid2d, W1, b1, W2, b2, W3, b3):
    blk = 4096
    grid = (BATCH // blk,)
    return pl.pallas_call(
        _mlp_block,
        grid=grid,
        in_specs=[
            pl.BlockSpec((blk, 128), lambda i: (i, 0)),
            pl.BlockSpec((blk, 1), lambda i: (i, 0)),
            pl.BlockSpec(W1.shape, lambda i: (0, 0)),
            pl.BlockSpec(b1.shape, lambda i: (0, 0)),
            pl.BlockSpec(W2.shape, lambda i: (0, 0)),
            pl.BlockSpec(b2.shape, lambda i: (0, 0)),
            pl.BlockSpec(W3.shape, lambda i: (0, 0)),
            pl.BlockSpec(b3.shape, lambda i: (0, 0)),
        ],
        out_specs=pl.BlockSpec((W3.shape[1], blk), lambda i: (0, i)),
        out_shape=jax.ShapeDtypeStruct((W3.shape[1], BATCH), jnp.float32),
    )(x, uid2d, W1, b1, W2, b2, W3, b3)


def kernel(user_id, table, W1, b1, W2, b2, W3, b3):
    uid = user_id.astype(jnp.int32)
    chunk = uid // _CHUNK_C
    r = uid % _CHUNK_C
    slot3d = (chunk * _QUARTER + r % _QUARTER).reshape(
        _NW, _N_CHUNKS, _IDX_CHUNK
    )
    packed = _tc_pack(table.T)
    rows = _sc_gather(packed, slot3d)
    outT = _tc_mlp(
        rows,
        uid.reshape(BATCH, 1),
        W1,
        b1.reshape(-1, 1),
        W2,
        b2.reshape(-1, 1),
        W3,
        b3.reshape(-1, 1),
    )
    return outT.T
